# Initial kernel scaffold; baseline (speedup 1.0000x reference)
import jax
import jax.numpy as jnp
from jax.experimental import pallas as pl


def kernel(x, token_table, pos_table):
    raise NotImplementedError("write your pallas kernel here")


# speedup vs baseline: 1.3960x; 1.3960x over previous
import jax
import jax.numpy as jnp
from jax import lax
from jax.experimental import pallas as pl
from jax.experimental.pallas import tpu as pltpu
from jax.experimental.pallas import tpu_sc as plsc

VOCAB = 1000000
MAXLEN = 200
EMBED = 32
BATCH = 4096

NC, NS, L = 2, 16, 16
NW = NC * NS
TOK_TOTAL = BATCH * MAXLEN
TOK_PER_W = TOK_TOTAL // NW
CH_SEQ = 4
CH_TOK = CH_SEQ * MAXLEN
NCH = TOK_PER_W // CH_TOK
SLEN = 100
NSTREAM = CH_TOK // SLEN
XROWS = TOK_TOTAL // SLEN


def _body(x_hbm, tok_hbm, posblk_hbm, out_hbm, idx_v, buf_v, pos_v, sem_g):
    wid = lax.axis_index("s") * NC + lax.axis_index("c")
    base_tok = wid * TOK_PER_W
    base_xrow = wid * (TOK_PER_W // SLEN)

    pltpu.sync_copy(posblk_hbm, pos_v)

    def chunk_body(c, carry):
        tok0 = base_tok + c * CH_TOK
        xrow0 = base_xrow + c * NSTREAM
        pltpu.sync_copy(x_hbm.at[pl.ds(xrow0, NSTREAM)], idx_v)

        def init_body(i, carry2):
            r = i * 8
            for dr in range(8):
                for h in range(2):
                    sl = pl.ds(h * L, L)
                    buf_v[r + dr, sl] = pos_v[r + dr, sl]
            return carry2

        lax.fori_loop(0, CH_TOK // 8, init_body, 0)

        descs = []
        for j in range(NSTREAM):
            descs.append(pltpu.async_copy(
                tok_hbm.at[idx_v.at[j]],
                buf_v.at[pl.ds(j * SLEN, SLEN)],
                sem_g, add=True))
        for d in descs:
            d.wait()

        pltpu.sync_copy(buf_v, out_hbm.at[pl.ds(tok0, CH_TOK)])
        return carry

    lax.fori_loop(0, NCH, chunk_body, 0)


def kernel(x, token_table, pos_table):
    x2 = x.astype(jnp.int32).reshape(XROWS, SLEN)
    posblk = jnp.tile(pos_table, (CH_SEQ, 1))
    mesh = plsc.VectorSubcoreMesh(core_axis_name="c", subcore_axis_name="s",
                                  num_cores=NC, num_subcores=NS)
    k = pl.kernel(
        _body,
        out_type=jax.ShapeDtypeStruct((TOK_TOTAL, EMBED), jnp.float32),
        mesh=mesh,
        scratch_types=[
            pltpu.VMEM((NSTREAM, SLEN), jnp.int32),
            pltpu.VMEM((CH_TOK, EMBED), jnp.float32),
            pltpu.VMEM((CH_TOK, EMBED), jnp.float32),
            pltpu.SemaphoreType.DMA,
        ],
        compiler_params=pltpu.CompilerParams(use_tc_tiling_on_sc=False),
    )
    out = k(x2, token_table, posblk)
    return out.reshape(BATCH, MAXLEN, EMBED)


# speedup vs baseline: 1.4599x; 1.0458x over previous
import jax
import jax.numpy as jnp
from jax import lax
from jax.experimental import pallas as pl
from jax.experimental.pallas import tpu as pltpu
from jax.experimental.pallas import tpu_sc as plsc

VOCAB = 1000000
MAXLEN = 200
EMBED = 32
BATCH = 4096

NC, NS, L = 2, 16, 16
NW = NC * NS
TOK_TOTAL = BATCH * MAXLEN
TOK_PER_W = TOK_TOTAL // NW
CH_SEQ = 4
CH_TOK = CH_SEQ * MAXLEN
NCH = TOK_PER_W // CH_TOK
NBUF = 2
SLEN = 100
NSTREAM = CH_TOK // SLEN
XROWS = TOK_TOTAL // SLEN


def _body(x_hbm, tok_hbm, posblk_hbm, out_hbm,
          idx0, idx1, buf0, buf1, pos_v,
          sem_i0, sem_i1, sem_g0, sem_g1, sem_s0, sem_s1):
    idxs = (idx0, idx1)
    bufs = (buf0, buf1)
    sem_i = (sem_i0, sem_i1)
    sem_g = (sem_g0, sem_g1)
    sem_s = (sem_s0, sem_s1)

    wid = lax.axis_index("s") * NC + lax.axis_index("c")
    base_tok = wid * TOK_PER_W
    base_xrow = wid * (TOK_PER_W // SLEN)

    pltpu.sync_copy(posblk_hbm, pos_v)

    for b in range(NBUF):
        pltpu.async_copy(
            x_hbm.at[pl.ds(base_xrow + b * NSTREAM, NSTREAM)],
            idxs[b], sem_i[b])

    def pair_body(it, carry):
        for b in range(NBUF):
            c = it * NBUF + b
            tok0 = base_tok + c * CH_TOK
            xrow0 = base_xrow + c * NSTREAM

            @pl.when(it > 0)
            def _():
                pltpu.make_async_copy(
                    bufs[b], out_hbm.at[pl.ds(tok0 - NBUF * CH_TOK, CH_TOK)],
                    sem_s[b]).wait()

            def init_body(i, carry2):
                r = i * 8
                for dr in range(8):
                    for h in range(2):
                        sl = pl.ds(h * L, L)
                        bufs[b][r + dr, sl] = pos_v[r + dr, sl]
                return carry2

            lax.fori_loop(0, CH_TOK // 8, init_body, 0)

            pltpu.make_async_copy(
                x_hbm.at[pl.ds(xrow0, NSTREAM)], idxs[b], sem_i[b]).wait()

            descs = []
            for j in range(NSTREAM):
                descs.append(pltpu.async_copy(
                    tok_hbm.at[idxs[b].at[j]],
                    bufs[b].at[pl.ds(j * SLEN, SLEN)],
                    sem_g[b], add=True))
            for d in descs:
                d.wait()

            pltpu.async_copy(bufs[b], out_hbm.at[pl.ds(tok0, CH_TOK)], sem_s[b])

            @pl.when(it < NCH // NBUF - 1)
            def _():
                pltpu.async_copy(
                    x_hbm.at[pl.ds(xrow0 + NBUF * NSTREAM, NSTREAM)],
                    idxs[b], sem_i[b])
        return carry

    lax.fori_loop(0, NCH // NBUF, pair_body, 0)

    for b in range(NBUF):
        tok0 = base_tok + (NCH - NBUF + b) * CH_TOK
        pltpu.make_async_copy(
            bufs[b], out_hbm.at[pl.ds(tok0, CH_TOK)], sem_s[b]).wait()


def kernel(x, token_table, pos_table):
    x2 = x.astype(jnp.int32).reshape(XROWS, SLEN)
    posblk = jnp.tile(pos_table, (CH_SEQ, 1))
    mesh = plsc.VectorSubcoreMesh(core_axis_name="c", subcore_axis_name="s",
                                  num_cores=NC, num_subcores=NS)
    k = pl.kernel(
        _body,
        out_type=jax.ShapeDtypeStruct((TOK_TOTAL, EMBED), jnp.float32),
        mesh=mesh,
        scratch_types=[
            pltpu.VMEM((NSTREAM, SLEN), jnp.int32),
            pltpu.VMEM((NSTREAM, SLEN), jnp.int32),
            pltpu.VMEM((CH_TOK, EMBED), jnp.float32),
            pltpu.VMEM((CH_TOK, EMBED), jnp.float32),
            pltpu.VMEM((CH_TOK, EMBED), jnp.float32),
            pltpu.SemaphoreType.DMA,
            pltpu.SemaphoreType.DMA,
            pltpu.SemaphoreType.DMA,
            pltpu.SemaphoreType.DMA,
            pltpu.SemaphoreType.DMA,
            pltpu.SemaphoreType.DMA,
        ],
        compiler_params=pltpu.CompilerParams(use_tc_tiling_on_sc=False),
    )
    out = k(x2, token_table, posblk)
    return out.reshape(BATCH, MAXLEN, EMBED)
